# Initial kernel scaffold; baseline (speedup 1.0000x reference)
import functools

import jax
import jax.numpy as jnp
from jax import lax
from jax.experimental import pallas as pl
from jax.experimental.pallas import tpu as pltpu
from jax.experimental.pallas import tpu_sc as plsc

NC = 2
NS = 16
NW = NC * NS

_G = 256


@functools.partial(functools.lru_cache, maxsize=None)
def _deg_kernel(E, Npad):
    C = 80
    per_w = E // NW
    iters = per_w // C
    rpt = Npad // NS
    mesh = plsc.VectorSubcoreMesh(core_axis_name="c", subcore_axis_name="s")

    @functools.partial(
        pl.kernel,
        out_type=jax.ShapeDtypeStruct((NC, Npad), jnp.float32),
        mesh=mesh,
        scratch_types=[
            pltpu.VMEM_SHARED((Npad,), jnp.float32),
            pltpu.VMEM((C,), jnp.int32),
            pltpu.VMEM((C,), jnp.float32),
            pltpu.VMEM((rpt,), jnp.float32),
        ],
    )
    def k(dst_hbm, out_hbm, acc, idx_v, ones_v, zbuf):
        cid = lax.axis_index("c")
        sid = lax.axis_index("s")
        wid = sid * NC + cid

        for i in range(C // 16):
            ones_v[pl.ds(i * 16, 16)] = jnp.ones((16,), jnp.float32)

        def zb(i, _):
            zbuf[pl.ds(i * 16, 16)] = jnp.zeros((16,), jnp.float32)
            return 0
        lax.fori_loop(0, rpt // 16, zb, 0)
        pltpu.sync_copy(zbuf, acc.at[pl.ds(sid * rpt, rpt)])
        plsc.subcore_barrier()

        base0 = wid * per_w

        def body(i, _):
            pltpu.sync_copy(dst_hbm.at[pl.ds(base0 + i * C, C)], idx_v)
            pltpu.sync_copy(ones_v, acc.at[idx_v], add=True)
            return 0
        lax.fori_loop(0, iters, body, 0)
        plsc.subcore_barrier()

        pltpu.sync_copy(acc.at[pl.ds(sid * rpt, rpt)],
                        out_hbm.at[cid, pl.ds(sid * rpt, rpt)])

    return k


@functools.partial(functools.lru_cache, maxsize=None)
def _edge_scatter_kernel(N, D, E):
    C = 80
    per_w = E // NW
    iters = per_w // C
    rpt = N // NS
    ZR = 125
    mesh = plsc.VectorSubcoreMesh(core_axis_name="c", subcore_axis_name="s")

    @functools.partial(
        pl.kernel,
        out_type=jax.ShapeDtypeStruct((NC, N, D), jnp.float32),
        mesh=mesh,
        scratch_types=[
            pltpu.VMEM_SHARED((N, D), jnp.float32),
            pltpu.VMEM((C,), jnp.int32),
            pltpu.VMEM((C,), jnp.int32),
            pltpu.VMEM((C, D), jnp.float32),
            pltpu.VMEM((ZR, D), jnp.float32),
            pltpu.SemaphoreType.DMA,
        ],
    )
    def k(h_hbm, src_hbm, dst_hbm, out_hbm, acc, sidx, didx, rows, zbuf, sem):
        cid = lax.axis_index("c")
        sid = lax.axis_index("s")
        wid = sid * NC + cid

        def zb(i, _):
            for c8 in range(D // 16):
                zbuf[i, pl.ds(c8 * 16, 16)] = jnp.zeros((16,), jnp.float32)
            return 0
        lax.fori_loop(0, ZR, zb, 0)
        for j in range(rpt // ZR):
            pltpu.sync_copy(zbuf, acc.at[pl.ds(sid * rpt + j * ZR, ZR)])
        plsc.subcore_barrier()

        base0 = wid * per_w

        def body(i, _):
            base = base0 + i * C
            pltpu.sync_copy(src_hbm.at[pl.ds(base, C)], sidx)
            pltpu.sync_copy(dst_hbm.at[pl.ds(base, C)], didx)
            pltpu.async_copy(h_hbm.at[sidx], rows, sem).wait()
            pltpu.sync_copy(rows, acc.at[didx], add=True)
            return 0
        lax.fori_loop(0, iters, body, 0)
        plsc.subcore_barrier()

        pltpu.sync_copy(acc.at[pl.ds(sid * rpt, rpt)],
                        out_hbm.at[cid, pl.ds(sid * rpt, rpt)])

    return k


@functools.partial(functools.lru_cache, maxsize=None)
def _a1_kernel(N, D, R):
    def body(degp_ref, x_ref, w_ref, dis_ref, hp_ref):
        deg = degp_ref[0] + degp_ref[1] + 1.0
        dis = lax.rsqrt(deg)
        dis_ref[...] = dis
        hp_ref[...] = jnp.dot(x_ref[...], w_ref[...],
                              preferred_element_type=jnp.float32) * dis

    return pl.pallas_call(
        body,
        grid=(N // R,),
        in_specs=[
            pl.BlockSpec((2, R, 1), lambda i: (0, i, 0)),
            pl.BlockSpec((R, D), lambda i: (i, 0)),
            pl.BlockSpec((D, D), lambda i: (0, 0)),
        ],
        out_specs=[
            pl.BlockSpec((R, 1), lambda i: (i, 0)),
            pl.BlockSpec((R, D), lambda i: (i, 0)),
        ],
        out_shape=[
            jax.ShapeDtypeStruct((N, 1), jnp.float32),
            jax.ShapeDtypeStruct((N, D), jnp.float32),
        ],
    )


@functools.partial(functools.lru_cache, maxsize=None)
def _a2_kernel(N, D, R):
    def body(p_ref, hp_ref, dis_ref, b_ref, w_ref, out_ref):
        dis = dis_ref[...]
        h = jnp.maximum(dis * (p_ref[0] + p_ref[1] + hp_ref[...]) + b_ref[...],
                        0.0)
        out_ref[...] = jnp.dot(h, w_ref[...],
                               preferred_element_type=jnp.float32) * dis

    return pl.pallas_call(
        body,
        grid=(N // R,),
        in_specs=[
            pl.BlockSpec((2, R, D), lambda i: (0, i, 0)),
            pl.BlockSpec((R, D), lambda i: (i, 0)),
            pl.BlockSpec((R, 1), lambda i: (i, 0)),
            pl.BlockSpec((1, D), lambda i: (0, 0)),
            pl.BlockSpec((D, D), lambda i: (0, 0)),
        ],
        out_specs=pl.BlockSpec((R, D), lambda i: (i, 0)),
        out_shape=jax.ShapeDtypeStruct((N, D), jnp.float32),
    )


@functools.partial(functools.lru_cache, maxsize=None)
def _a3_kernel(N, D, G, L, R):
    steps = N // R

    def body(p_ref, hp_ref, dis_ref, b_ref, batch_ref,
             mw1_ref, mb1_ref, mw2_ref, mb2_ref,
             out_ref, sums_ref, cnt_ref):
        i = pl.program_id(0)
        dis = dis_ref[...]
        h = jnp.maximum(dis * (p_ref[0] + p_ref[1] + hp_ref[...]) + b_ref[...],
                        0.0)
        gids = lax.broadcasted_iota(jnp.int32, (R, G), 1)
        oh = (batch_ref[...] == gids).astype(jnp.float32)
        contrib = lax.dot_general(oh, h, (((0,), (0,)), ((), ())),
                                  preferred_element_type=jnp.float32)
        cnt_c = lax.dot_general(oh, jnp.ones((R, 1), jnp.float32),
                                (((0,), (0,)), ((), ())),
                                preferred_element_type=jnp.float32)

        @pl.when(i == 0)
        def _():
            sums_ref[...] = jnp.zeros_like(sums_ref)
            cnt_ref[...] = jnp.zeros_like(cnt_ref)

        sums_ref[...] += contrib
        cnt_ref[...] += cnt_c

        @pl.when(i == steps - 1)
        def _():
            pooled = sums_ref[...] / jnp.maximum(cnt_ref[...], 1.0)
            z = jnp.maximum(jnp.dot(pooled, mw1_ref[...],
                                    preferred_element_type=jnp.float32)
                            + mb1_ref[...], 0.0)
            out_ref[...] = jnp.dot(z, mw2_ref[...],
                                   preferred_element_type=jnp.float32) \
                           + mb2_ref[...]

    return pl.pallas_call(
        body,
        grid=(steps,),
        in_specs=[
            pl.BlockSpec((2, R, D), lambda i: (0, i, 0)),
            pl.BlockSpec((R, D), lambda i: (i, 0)),
            pl.BlockSpec((R, 1), lambda i: (i, 0)),
            pl.BlockSpec((1, D), lambda i: (0, 0)),
            pl.BlockSpec((R, 1), lambda i: (i, 0)),
            pl.BlockSpec((D, D), lambda i: (0, 0)),
            pl.BlockSpec((1, D), lambda i: (0, 0)),
            pl.BlockSpec((D, L), lambda i: (0, 0)),
            pl.BlockSpec((1, L), lambda i: (0, 0)),
        ],
        out_specs=pl.BlockSpec((G, L), lambda i: (0, 0)),
        out_shape=jax.ShapeDtypeStruct((G, L), jnp.float32),
        scratch_shapes=[
            pltpu.VMEM((G, D), jnp.float32),
            pltpu.VMEM((G, 1), jnp.float32),
        ],
    )


def kernel(x, edge_index, batch, W1, b1, W2, b2, mW1, mb1, mW2, mb2):
    N, D = x.shape
    E = edge_index.shape[1]
    L = mW2.shape[0]
    G = _G
    R = 2000

    src = edge_index[0]
    dst = edge_index[1]

    Npad = NS * ((N + NS * 16 - 1) // (NS * 16)) * 16
    degp = _deg_kernel(E, Npad)(dst)
    degp = degp[:, :N].reshape(NC, N, 1)

    dis, h1p = _a1_kernel(N, D, R)(degp, x, W1.T)
    p1 = _edge_scatter_kernel(N, D, E)(h1p, src, dst)
    h2p = _a2_kernel(N, D, R)(p1, h1p, dis, b1.reshape(1, D), W2.T)
    p2 = _edge_scatter_kernel(N, D, E)(h2p, src, dst)
    logits = _a3_kernel(N, D, G, L, R)(
        p2, h2p, dis, b2.reshape(1, D), batch.reshape(N, 1),
        mW1.T, mb1.reshape(1, D), mW2.T, mb2.reshape(1, L))
    return logits


# speedup vs baseline: 13.3963x; 13.3963x over previous
import functools

import jax
import jax.numpy as jnp
from jax import lax
from jax.experimental import pallas as pl
from jax.experimental.pallas import tpu as pltpu
from jax.experimental.pallas import tpu_sc as plsc

NC = 2
NS = 16
NW = NC * NS

_G = 256


@functools.lru_cache(maxsize=None)
def _deg_kernel(E, Npad):
    C = 80
    per_w = E // NW
    iters = per_w // C
    rpt = Npad // NS
    mesh = plsc.VectorSubcoreMesh(core_axis_name="c", subcore_axis_name="s")

    @functools.partial(
        pl.kernel,
        out_type=jax.ShapeDtypeStruct((NC, Npad), jnp.float32),
        mesh=mesh,
        scratch_types=[
            pltpu.VMEM_SHARED((Npad,), jnp.float32),
            pltpu.VMEM((C,), jnp.int32),
            pltpu.VMEM((C,), jnp.float32),
            pltpu.VMEM((rpt,), jnp.float32),
        ],
    )
    def k(dst_hbm, out_hbm, acc, idx_v, ones_v, zbuf):
        cid = lax.axis_index("c")
        sid = lax.axis_index("s")
        wid = sid * NC + cid

        for i in range(C // 16):
            ones_v[pl.ds(i * 16, 16)] = jnp.ones((16,), jnp.float32)

        def zb(i, _):
            zbuf[pl.ds(i * 16, 16)] = jnp.zeros((16,), jnp.float32)
            return 0
        lax.fori_loop(0, rpt // 16, zb, 0)
        pltpu.sync_copy(zbuf, acc.at[pl.ds(sid * rpt, rpt)])
        plsc.subcore_barrier()

        base0 = wid * per_w

        def body(i, _):
            pltpu.sync_copy(dst_hbm.at[pl.ds(base0 + i * C, C)], idx_v)
            pltpu.sync_copy(ones_v, acc.at[idx_v], add=True)
            return 0
        lax.fori_loop(0, iters, body, 0)
        plsc.subcore_barrier()

        pltpu.sync_copy(acc.at[pl.ds(sid * rpt, rpt)],
                        out_hbm.at[cid, pl.ds(sid * rpt, rpt)])

    return k


@functools.lru_cache(maxsize=None)
def _edge_scatter_kernel(N, D, E):
    C = 80
    per_w = E // NW
    iters = per_w // C
    rpt = (N // NS) // 8 * 8
    tail = N - NS * rpt
    ZR = 104
    assert rpt % ZR == 0 and tail % 8 == 0 and tail <= ZR
    mesh = plsc.VectorSubcoreMesh(core_axis_name="c", subcore_axis_name="s")

    @functools.partial(
        pl.kernel,
        out_type=jax.ShapeDtypeStruct((NC, N, D), jnp.float32),
        mesh=mesh,
        scratch_types=[
            pltpu.VMEM_SHARED((N, D), jnp.float32),
            pltpu.VMEM((C,), jnp.int32),
            pltpu.VMEM((C,), jnp.int32),
            pltpu.VMEM((C, D), jnp.float32),
            pltpu.VMEM((ZR, D), jnp.float32),
            pltpu.SemaphoreType.DMA,
        ],
    )
    def k(h_hbm, src_hbm, dst_hbm, out_hbm, acc, sidx, didx, rows, zbuf, sem):
        cid = lax.axis_index("c")
        sid = lax.axis_index("s")
        wid = sid * NC + cid

        def zb(i, _):
            for c8 in range(D // 16):
                zbuf[i, pl.ds(c8 * 16, 16)] = jnp.zeros((16,), jnp.float32)
            return 0
        lax.fori_loop(0, ZR, zb, 0)
        for j in range(rpt // ZR):
            pltpu.sync_copy(zbuf, acc.at[pl.ds(sid * rpt + j * ZR, ZR)])

        @pl.when(sid == NS - 1)
        def _():
            pltpu.sync_copy(zbuf.at[pl.ds(0, tail)],
                            acc.at[pl.ds(NS * rpt, tail)])
        plsc.subcore_barrier()

        base0 = wid * per_w

        def body(i, _):
            base = base0 + i * C
            pltpu.sync_copy(src_hbm.at[pl.ds(base, C)], sidx)
            pltpu.sync_copy(dst_hbm.at[pl.ds(base, C)], didx)
            pltpu.async_copy(h_hbm.at[sidx], rows, sem).wait()
            pltpu.sync_copy(rows, acc.at[didx], add=True)
            return 0
        lax.fori_loop(0, iters, body, 0)
        plsc.subcore_barrier()

        pltpu.sync_copy(acc.at[pl.ds(sid * rpt, rpt)],
                        out_hbm.at[cid, pl.ds(sid * rpt, rpt)])

        @pl.when(sid == NS - 1)
        def _():
            pltpu.sync_copy(acc.at[pl.ds(NS * rpt, tail)],
                            out_hbm.at[cid, pl.ds(NS * rpt, tail)])

    return k


@functools.lru_cache(maxsize=None)
def _a1_kernel(N, D, R):
    def body(degp_ref, x_ref, w_ref, dis_ref, hp_ref):
        deg = degp_ref[0] + degp_ref[1] + 1.0
        dis = lax.rsqrt(deg)
        dis_ref[...] = dis
        hp_ref[...] = jnp.dot(x_ref[...], w_ref[...],
                              preferred_element_type=jnp.float32) * dis

    return pl.pallas_call(
        body,
        grid=(N // R,),
        in_specs=[
            pl.BlockSpec((2, R, 1), lambda i: (0, i, 0)),
            pl.BlockSpec((R, D), lambda i: (i, 0)),
            pl.BlockSpec((D, D), lambda i: (0, 0)),
        ],
        out_specs=[
            pl.BlockSpec((R, 1), lambda i: (i, 0)),
            pl.BlockSpec((R, D), lambda i: (i, 0)),
        ],
        out_shape=[
            jax.ShapeDtypeStruct((N, 1), jnp.float32),
            jax.ShapeDtypeStruct((N, D), jnp.float32),
        ],
    )


@functools.lru_cache(maxsize=None)
def _a2_kernel(N, D, R):
    def body(p_ref, hp_ref, dis_ref, b_ref, w_ref, out_ref):
        dis = dis_ref[...]
        h = jnp.maximum(dis * (p_ref[0] + p_ref[1] + hp_ref[...]) + b_ref[...],
                        0.0)
        out_ref[...] = jnp.dot(h, w_ref[...],
                               preferred_element_type=jnp.float32) * dis

    return pl.pallas_call(
        body,
        grid=(N // R,),
        in_specs=[
            pl.BlockSpec((2, R, D), lambda i: (0, i, 0)),
            pl.BlockSpec((R, D), lambda i: (i, 0)),
            pl.BlockSpec((R, 1), lambda i: (i, 0)),
            pl.BlockSpec((1, D), lambda i: (0, 0)),
            pl.BlockSpec((D, D), lambda i: (0, 0)),
        ],
        out_specs=pl.BlockSpec((R, D), lambda i: (i, 0)),
        out_shape=jax.ShapeDtypeStruct((N, D), jnp.float32),
    )


@functools.lru_cache(maxsize=None)
def _a3_kernel(N, D, G, L, R):
    steps = N // R

    def body(p_ref, hp_ref, dis_ref, b_ref, batch_ref,
             mw1_ref, mb1_ref, mw2_ref, mb2_ref,
             out_ref, sums_ref, cnt_ref):
        i = pl.program_id(0)
        dis = dis_ref[...]
        h = jnp.maximum(dis * (p_ref[0] + p_ref[1] + hp_ref[...]) + b_ref[...],
                        0.0)
        gids = lax.broadcasted_iota(jnp.int32, (R, G), 1)
        oh = (batch_ref[...] == gids).astype(jnp.float32)
        contrib = lax.dot_general(oh, h, (((0,), (0,)), ((), ())),
                                  preferred_element_type=jnp.float32)
        cnt_c = lax.dot_general(oh, jnp.ones((R, 1), jnp.float32),
                                (((0,), (0,)), ((), ())),
                                preferred_element_type=jnp.float32)

        @pl.when(i == 0)
        def _():
            sums_ref[...] = jnp.zeros_like(sums_ref)
            cnt_ref[...] = jnp.zeros_like(cnt_ref)

        sums_ref[...] += contrib
        cnt_ref[...] += cnt_c

        @pl.when(i == steps - 1)
        def _():
            pooled = sums_ref[...] / jnp.maximum(cnt_ref[...], 1.0)
            z = jnp.maximum(jnp.dot(pooled, mw1_ref[...],
                                    preferred_element_type=jnp.float32)
                            + mb1_ref[...], 0.0)
            out_ref[...] = jnp.dot(z, mw2_ref[...],
                                   preferred_element_type=jnp.float32) \
                           + mb2_ref[...]

    return pl.pallas_call(
        body,
        grid=(steps,),
        in_specs=[
            pl.BlockSpec((2, R, D), lambda i: (0, i, 0)),
            pl.BlockSpec((R, D), lambda i: (i, 0)),
            pl.BlockSpec((R, 1), lambda i: (i, 0)),
            pl.BlockSpec((1, D), lambda i: (0, 0)),
            pl.BlockSpec((R, 1), lambda i: (i, 0)),
            pl.BlockSpec((D, D), lambda i: (0, 0)),
            pl.BlockSpec((1, D), lambda i: (0, 0)),
            pl.BlockSpec((D, L), lambda i: (0, 0)),
            pl.BlockSpec((1, L), lambda i: (0, 0)),
        ],
        out_specs=pl.BlockSpec((G, L), lambda i: (0, 0)),
        out_shape=jax.ShapeDtypeStruct((G, L), jnp.float32),
        scratch_shapes=[
            pltpu.VMEM((G, D), jnp.float32),
            pltpu.VMEM((G, 1), jnp.float32),
        ],
    )


def kernel(x, edge_index, batch, W1, b1, W2, b2, mW1, mb1, mW2, mb2):
    N, D = x.shape
    E = edge_index.shape[1]
    L = mW2.shape[0]
    G = _G
    R = 2000

    src = edge_index[0]
    dst = edge_index[1]

    Npad = NS * ((N + NS * 16 - 1) // (NS * 16)) * 16
    degp = _deg_kernel(E, Npad)(dst)
    degp = degp[:, :N].reshape(NC, N, 1)

    dis, h1p = _a1_kernel(N, D, R)(degp, x, W1.T)
    p1 = _edge_scatter_kernel(N, D, E)(h1p, src, dst)
    h2p = _a2_kernel(N, D, R)(p1, h1p, dis, b1.reshape(1, D), W2.T)
    p2 = _edge_scatter_kernel(N, D, E)(h2p, src, dst)
    logits = _a3_kernel(N, D, G, L, R)(
        p2, h2p, dis, b2.reshape(1, D), batch.reshape(N, 1),
        mW1.T, mb1.reshape(1, D), mW2.T, mb2.reshape(1, L))
    return logits


# speedup vs baseline: 29.4320x; 2.1970x over previous
import functools

import jax
import jax.numpy as jnp
from jax import lax
from jax.experimental import pallas as pl
from jax.experimental.pallas import tpu as pltpu
from jax.experimental.pallas import tpu_sc as plsc

NC = 2
NS = 16
NW = NC * NS

_G = 256


@functools.lru_cache(maxsize=None)
def _deg_kernel(E, Npad):
    C = 80
    per_w = E // NW
    iters = per_w // C
    NB = 4
    rpt = Npad // NS
    mesh = plsc.VectorSubcoreMesh(core_axis_name="c", subcore_axis_name="s")

    @functools.partial(
        pl.kernel,
        out_type=jax.ShapeDtypeStruct((NC, Npad), jnp.float32),
        mesh=mesh,
        scratch_types=[
            pltpu.VMEM_SHARED((Npad,), jnp.float32),
            pltpu.VMEM((iters, C), jnp.int32),
            pltpu.VMEM((C,), jnp.float32),
            pltpu.VMEM((rpt,), jnp.float32),
        ] + [pltpu.SemaphoreType.DMA] * NB,
    )
    def k(dst_hbm, out_hbm, acc, idx_v, ones_v, zbuf, *sems):
        cid = lax.axis_index("c")
        sid = lax.axis_index("s")
        wid = sid * NC + cid

        for i in range(C // 16):
            ones_v[pl.ds(i * 16, 16)] = jnp.ones((16,), jnp.float32)

        def zb(i, _):
            zbuf[pl.ds(i * 16, 16)] = jnp.zeros((16,), jnp.float32)
            return 0
        lax.fori_loop(0, rpt // 16, zb, 0)
        pltpu.sync_copy(zbuf, acc.at[pl.ds(sid * rpt, rpt)])
        pltpu.sync_copy(dst_hbm.at[wid], idx_v)
        plsc.subcore_barrier()

        def scat(k_, b):
            return pltpu.make_async_copy(ones_v, acc.at[idx_v.at[k_]],
                                         sems[b])

        for b in range(NB):
            scat(b, b).start(add=True)

        def body(j, _):
            for b in range(NB):
                scat(NB * j + b, b).wait()
                scat(NB * (j + 1) + b, b).start(add=True)
            return 0
        lax.fori_loop(0, iters // NB - 1, body, 0)
        for b in range(iters // NB * NB, iters):
            scat(b, b - iters // NB * NB).wait()
            scat(b, b % NB).start(add=True)
        for b in range(NB):
            scat(0, b).wait()
        plsc.subcore_barrier()

        pltpu.sync_copy(acc.at[pl.ds(sid * rpt, rpt)],
                        out_hbm.at[cid, pl.ds(sid * rpt, rpt)])

    return k


@functools.lru_cache(maxsize=None)
def _edge_scatter_kernel(N, D, E, C, NB):
    H = D // NC
    iters = E // NS // C
    nfull = iters // NB
    rem = iters - NB * nfull
    rpt = (N // NS) // 8 * 8
    tail = N - NS * rpt
    ZR = 48
    assert rpt % ZR == 0 and tail % 8 == 0 and tail <= ZR
    mesh = plsc.VectorSubcoreMesh(core_axis_name="c", subcore_axis_name="s")

    @functools.partial(
        pl.kernel,
        out_type=jax.ShapeDtypeStruct((NC, N, H), jnp.float32),
        mesh=mesh,
        scratch_types=[
            pltpu.VMEM_SHARED((N, H), jnp.float32),
            pltpu.VMEM((iters, C), jnp.int32),
            pltpu.VMEM((iters, C), jnp.int32),
            pltpu.VMEM((ZR, H), jnp.float32),
        ] + [pltpu.VMEM((C, H), jnp.float32)] * NB
          + [pltpu.SemaphoreType.DMA] * (2 * NB + 1),
        compiler_params=pltpu.CompilerParams(use_tc_tiling_on_sc=False),
    )
    def k(h_hbm, src_hbm, dst_hbm, out_hbm, acc, sidx, didx, zbuf, *rest):
        rows = rest[:NB]
        gsem = rest[NB:2 * NB]
        ssem = rest[2 * NB:3 * NB]
        zsem = rest[3 * NB]
        cid = lax.axis_index("c")
        sid = lax.axis_index("s")

        def zb(i, _):
            for c16 in range(H // 16):
                zbuf[i, pl.ds(c16 * 16, 16)] = jnp.zeros((16,), jnp.float32)
            return 0
        lax.fori_loop(0, ZR, zb, 0)
        nz = rpt // ZR
        for j in range(nz):
            pltpu.make_async_copy(
                zbuf, acc.at[pl.ds(sid * rpt + j * ZR, ZR)], zsem).start()
        pltpu.sync_copy(src_hbm.at[sid], sidx)
        pltpu.sync_copy(dst_hbm.at[sid], didx)
        for j in range(nz):
            pltpu.make_async_copy(
                zbuf, acc.at[pl.ds(sid * rpt, ZR)], zsem).wait()

        @pl.when(sid == NS - 1)
        def _():
            pltpu.sync_copy(zbuf.at[pl.ds(0, tail)],
                            acc.at[pl.ds(NS * rpt, tail)])
        plsc.subcore_barrier()

        def gath(k_, b):
            return pltpu.make_async_copy(h_hbm.at[cid].at[sidx.at[k_]],
                                         rows[b], gsem[b])

        def scat(k_, b):
            return pltpu.make_async_copy(rows[b], acc.at[didx.at[k_]],
                                         ssem[b])

        for b in range(NB):
            gath(b, b).start()

        def body(j, _):
            for b in range(NB):
                k_ = NB * j + b
                gath(k_, b).wait()
                scat(k_, b).start(add=True)
            for b in range(NB):
                scat(0, b).wait()
                gath(NB * (j + 1) + b, b).start()
            return 0
        lax.fori_loop(0, nfull - 1, body, 0)

        for b in range(NB):
            k_ = NB * (nfull - 1) + b
            gath(k_, b).wait()
            scat(k_, b).start(add=True)
        for t in range(rem):
            b = t % NB
            scat(0, b).wait()
            gath(NB * nfull + t, b).start()
            gath(0, b).wait()
            scat(NB * nfull + t, b).start(add=True)
        for b in range(NB):
            scat(0, b).wait()
        plsc.subcore_barrier()

        pltpu.sync_copy(acc.at[pl.ds(sid * rpt, rpt)],
                        out_hbm.at[cid, pl.ds(sid * rpt, rpt)])

        @pl.when(sid == NS - 1)
        def _():
            pltpu.sync_copy(acc.at[pl.ds(NS * rpt, tail)],
                            out_hbm.at[cid, pl.ds(NS * rpt, tail)])

    return k


def _split_cols(hp):
    h = hp.shape[-1] // 2
    return hp[:, :h], hp[:, h:]


@functools.lru_cache(maxsize=None)
def _a1_kernel(N, D, R):
    H = D // NC

    def body(degp_ref, x_ref, w_ref, dis_ref, hp_ref):
        deg = degp_ref[0] + degp_ref[1] + 1.0
        dis = lax.rsqrt(deg)
        dis_ref[...] = dis
        hp = jnp.dot(x_ref[...], w_ref[...],
                     preferred_element_type=jnp.float32) * dis
        left, right = _split_cols(hp)
        hp_ref[0] = left
        hp_ref[1] = right

    return pl.pallas_call(
        body,
        grid=(N // R,),
        in_specs=[
            pl.BlockSpec((2, R, 1), lambda i: (0, i, 0)),
            pl.BlockSpec((R, D), lambda i: (i, 0)),
            pl.BlockSpec((D, D), lambda i: (0, 0)),
        ],
        out_specs=[
            pl.BlockSpec((R, 1), lambda i: (i, 0)),
            pl.BlockSpec((2, R, H), lambda i: (0, i, 0)),
        ],
        out_shape=[
            jax.ShapeDtypeStruct((N, 1), jnp.float32),
            jax.ShapeDtypeStruct((2, N, H), jnp.float32),
        ],
    )


@functools.lru_cache(maxsize=None)
def _a2_kernel(N, D, R):
    H = D // NC

    def body(p_ref, hp_ref, dis_ref, b_ref, w_ref, out_ref):
        dis = dis_ref[...]
        msg = jnp.concatenate([p_ref[0] + hp_ref[0], p_ref[1] + hp_ref[1]],
                              axis=1)
        h = jnp.maximum(dis * msg + b_ref[...], 0.0)
        hp = jnp.dot(h, w_ref[...],
                     preferred_element_type=jnp.float32) * dis
        left, right = _split_cols(hp)
        out_ref[0] = left
        out_ref[1] = right

    return pl.pallas_call(
        body,
        grid=(N // R,),
        in_specs=[
            pl.BlockSpec((2, R, H), lambda i: (0, i, 0)),
            pl.BlockSpec((2, R, H), lambda i: (0, i, 0)),
            pl.BlockSpec((R, 1), lambda i: (i, 0)),
            pl.BlockSpec((1, D), lambda i: (0, 0)),
            pl.BlockSpec((D, D), lambda i: (0, 0)),
        ],
        out_specs=pl.BlockSpec((2, R, H), lambda i: (0, i, 0)),
        out_shape=jax.ShapeDtypeStruct((2, N, H), jnp.float32),
    )


@functools.lru_cache(maxsize=None)
def _a3_kernel(N, D, G, L, R):
    steps = N // R
    H = D // NC

    def body(p_ref, hp_ref, dis_ref, b_ref, batch_ref,
             mw1_ref, mb1_ref, mw2_ref, mb2_ref,
             out_ref, sums_ref, cnt_ref):
        i = pl.program_id(0)
        dis = dis_ref[...]
        msg = jnp.concatenate([p_ref[0] + hp_ref[0], p_ref[1] + hp_ref[1]],
                              axis=1)
        h = jnp.maximum(dis * msg + b_ref[...], 0.0)
        gids = lax.broadcasted_iota(jnp.int32, (R, G), 1)
        oh = (batch_ref[...] == gids).astype(jnp.float32)
        contrib = lax.dot_general(oh, h, (((0,), (0,)), ((), ())),
                                  preferred_element_type=jnp.float32)
        cnt_c = lax.dot_general(oh, jnp.ones((R, 1), jnp.float32),
                                (((0,), (0,)), ((), ())),
                                preferred_element_type=jnp.float32)

        @pl.when(i == 0)
        def _():
            sums_ref[...] = jnp.zeros_like(sums_ref)
            cnt_ref[...] = jnp.zeros_like(cnt_ref)

        sums_ref[...] += contrib
        cnt_ref[...] += cnt_c

        @pl.when(i == steps - 1)
        def _():
            pooled = sums_ref[...] / jnp.maximum(cnt_ref[...], 1.0)
            z = jnp.maximum(jnp.dot(pooled, mw1_ref[...],
                                    preferred_element_type=jnp.float32)
                            + mb1_ref[...], 0.0)
            out_ref[...] = jnp.dot(z, mw2_ref[...],
                                   preferred_element_type=jnp.float32) \
                           + mb2_ref[...]

    return pl.pallas_call(
        body,
        grid=(steps,),
        in_specs=[
            pl.BlockSpec((2, R, H), lambda i: (0, i, 0)),
            pl.BlockSpec((2, R, H), lambda i: (0, i, 0)),
            pl.BlockSpec((R, 1), lambda i: (i, 0)),
            pl.BlockSpec((1, D), lambda i: (0, 0)),
            pl.BlockSpec((R, 1), lambda i: (i, 0)),
            pl.BlockSpec((D, D), lambda i: (0, 0)),
            pl.BlockSpec((1, D), lambda i: (0, 0)),
            pl.BlockSpec((D, L), lambda i: (0, 0)),
            pl.BlockSpec((1, L), lambda i: (0, 0)),
        ],
        out_specs=pl.BlockSpec((G, L), lambda i: (0, 0)),
        out_shape=jax.ShapeDtypeStruct((G, L), jnp.float32),
        scratch_shapes=[
            pltpu.VMEM((G, D), jnp.float32),
            pltpu.VMEM((G, 1), jnp.float32),
        ],
    )


def kernel(x, edge_index, batch, W1, b1, W2, b2, mW1, mb1, mW2, mb2):
    N, D = x.shape
    E = edge_index.shape[1]
    L = mW2.shape[0]
    G = _G
    R = 2000

    C = 100
    NB = 4
    dstW = edge_index[1].reshape(NW, -1, 80)
    srcS = edge_index[0].reshape(NS, -1, C)
    dstS = edge_index[1].reshape(NS, -1, C)

    Npad = NS * ((N + NS * 16 - 1) // (NS * 16)) * 16
    degp = _deg_kernel(E, Npad)(dstW)
    degp = degp[:, :N].reshape(NC, N, 1)

    dis, h1p = _a1_kernel(N, D, R)(degp, x, W1.T)
    p1 = _edge_scatter_kernel(N, D, E, C, NB)(h1p, srcS, dstS)
    h2p = _a2_kernel(N, D, R)(p1, h1p, dis, b1.reshape(1, D), W2.T)
    p2 = _edge_scatter_kernel(N, D, E, C, NB)(h2p, srcS, dstS)
    logits = _a3_kernel(N, D, G, L, R)(
        p2, h2p, dis, b2.reshape(1, D), batch.reshape(N, 1),
        mW1.T, mb1.reshape(1, D), mW2.T, mb2.reshape(1, L))
    return logits
